# Initial kernel scaffold; baseline (speedup 1.0000x reference)
import jax
import jax.numpy as jnp
from jax.experimental import pallas as pl


def kernel(nodes, segment_ids):
    raise NotImplementedError("write your pallas kernel here")


# speedup vs baseline: 3.5889x; 3.5889x over previous
import jax
import jax.numpy as jnp
from jax import lax
from jax.experimental import pallas as pl
from jax.experimental.pallas import tpu as pltpu
from jax.experimental.pallas import tpu_sc as plsc

_N = 100000
_F = 128
_S = 128
_CH = 128
_NCH = -(-_N // _CH)
_NS = 16
_NC = 2
_CPT = -(-_NCH // _NS)
_FH = _F // _NC
_LAST = _NCH - 1
_TAIL = _N - _LAST * _CH


def _body(nodes_hbm, ids_hbm, zeros_hbm, out_hbm, ids_v, rows_v, acc_sh):
    ci = lax.axis_index("c")
    si = lax.axis_index("s")
    feat0 = ci * _FH
    start = si * _CPT
    stop = jnp.minimum(start + _CPT, _NCH)

    pltpu.sync_copy(ids_hbm.at[pl.ds(start, _CPT), :], ids_v)

    @pl.when(si == 0)
    def _():
        pltpu.sync_copy(zeros_hbm, acc_sh)

    plsc.subcore_barrier()

    def step(j, carry):
        c = start + j
        row0 = c * _CH

        @pl.when(c != _LAST)
        def _():
            pltpu.sync_copy(
                nodes_hbm.at[pl.ds(row0, _CH), pl.ds(feat0, _FH)], rows_v
            )

        @pl.when(c == _LAST)
        def _():
            pltpu.sync_copy(
                nodes_hbm.at[pl.ds(row0, _TAIL), pl.ds(feat0, _FH)],
                rows_v.at[pl.ds(0, _TAIL), :],
            )
            pltpu.sync_copy(
                zeros_hbm.at[pl.ds(0, _CH - _TAIL), :],
                rows_v.at[pl.ds(_TAIL, _CH - _TAIL), :],
            )

        pltpu.sync_copy(rows_v, acc_sh.at[ids_v.at[j]], add=True)
        return carry

    lax.fori_loop(0, stop - start, step, 0)

    plsc.subcore_barrier()

    @pl.when(si == 0)
    def _():
        pltpu.sync_copy(acc_sh, out_hbm.at[:, pl.ds(feat0, _FH)])


def kernel(nodes, segment_ids):
    ids_pad = (
        jnp.zeros((_NS * _CPT * _CH,), jnp.int32).at[:_N].set(segment_ids)
    )
    ids2d = ids_pad.reshape(_NS * _CPT, _CH)
    zeros = jnp.zeros((_S, _FH), jnp.float32)
    mesh = plsc.VectorSubcoreMesh(core_axis_name="c", subcore_axis_name="s")
    f = pl.kernel(
        _body,
        out_type=jax.ShapeDtypeStruct((_S, _F), jnp.float32),
        mesh=mesh,
        scratch_types=[
            pltpu.VMEM((_CPT, _CH), jnp.int32),
            pltpu.VMEM((_CH, _FH), jnp.float32),
            pltpu.VMEM_SHARED((_S, _FH), jnp.float32),
        ],
        compiler_params=pltpu.CompilerParams(use_tc_tiling_on_sc=False),
    )
    return f(nodes, ids2d, zeros)


# speedup vs baseline: 4.0212x; 1.1205x over previous
import jax
import jax.numpy as jnp
from jax import lax
from jax.experimental import pallas as pl
from jax.experimental.pallas import tpu as pltpu
from jax.experimental.pallas import tpu_sc as plsc

_N = 100000
_F = 128
_S = 128
_CH = 128
_NCH = -(-_N // _CH)
_NS = 16
_NC = 2
_CPT = -(-_NCH // _NS)
_FH = _F // _NC
_LAST = _NCH - 1
_TAIL = _N - _LAST * _CH


def _body(nodes_hbm, ids_hbm, zeros_hbm, out_hbm, ids_v, rows_v, acc_sh,
          sem0, sem1):
    ci = lax.axis_index("c")
    si = lax.axis_index("s")
    feat0 = ci * _FH
    start = si * _CPT
    sems = (sem0, sem1)

    pltpu.sync_copy(ids_hbm.at[pl.ds(start, _CPT), :], ids_v)

    @pl.when(si == 0)
    def _():
        pltpu.sync_copy(zeros_hbm, acc_sh)

    plsc.subcore_barrier()

    def issue_load(j, buf):
        c = start + j
        row0 = c * _CH

        @pl.when(jnp.logical_and(c < _NCH, c != _LAST))
        def _():
            pltpu.async_copy(
                nodes_hbm.at[pl.ds(row0, _CH), pl.ds(feat0, _FH)],
                rows_v.at[buf],
                sems[buf],
            )

        @pl.when(c == _LAST)
        def _():
            pltpu.async_copy(
                nodes_hbm.at[pl.ds(row0, _TAIL), pl.ds(feat0, _FH)],
                rows_v.at[buf, pl.ds(0, _TAIL), :],
                sems[buf],
            )
            pltpu.async_copy(
                zeros_hbm.at[pl.ds(0, _CH - _TAIL), :],
                rows_v.at[buf, pl.ds(_TAIL, _CH - _TAIL), :],
                sems[buf],
            )

    issue_load(0, 0)
    for j in range(_CPT):
        buf = j % 2
        c = start + j
        if j + 1 < _CPT:
            issue_load(j + 1, 1 - buf)

        @pl.when(c < _NCH)
        def _():
            pltpu.make_async_copy(
                nodes_hbm.at[pl.ds(0, _CH), pl.ds(0, _FH)],
                rows_v.at[buf],
                sems[buf],
            ).wait()
            pltpu.sync_copy(rows_v.at[buf], acc_sh.at[ids_v.at[j]], add=True)

    plsc.subcore_barrier()

    @pl.when(si == 0)
    def _():
        pltpu.sync_copy(acc_sh, out_hbm.at[:, pl.ds(feat0, _FH)])


def kernel(nodes, segment_ids):
    ids_pad = (
        jnp.zeros((_NS * _CPT * _CH,), jnp.int32).at[:_N].set(segment_ids)
    )
    ids2d = ids_pad.reshape(_NS * _CPT, _CH)
    zeros = jnp.zeros((_S, _FH), jnp.float32)
    mesh = plsc.VectorSubcoreMesh(core_axis_name="c", subcore_axis_name="s")
    f = pl.kernel(
        _body,
        out_type=jax.ShapeDtypeStruct((_S, _F), jnp.float32),
        mesh=mesh,
        scratch_types=[
            pltpu.VMEM((_CPT, _CH), jnp.int32),
            pltpu.VMEM((2, _CH, _FH), jnp.float32),
            pltpu.VMEM_SHARED((_S, _FH), jnp.float32),
            pltpu.SemaphoreType.DMA,
            pltpu.SemaphoreType.DMA,
        ],
        compiler_params=pltpu.CompilerParams(use_tc_tiling_on_sc=False),
    )
    return f(nodes, ids2d, zeros)


# speedup vs baseline: 4.1564x; 1.0336x over previous
import jax
import jax.numpy as jnp
from jax import lax
from jax.experimental import pallas as pl
from jax.experimental.pallas import tpu as pltpu
from jax.experimental.pallas import tpu_sc as plsc

_N = 100000
_F = 128
_S = 128
_CH = 125
_CW = 128
_NS = 16
_NC = 2
_CPT = 50
_FH = _F // _NC
_D = 8
_E = 4


def _body(nodes_hbm, ids_hbm, zeros_hbm, out_hbm, ids_v, rows_v, acc_sh,
          load_sem, scat_sem):
    ci = lax.axis_index("c")
    si = lax.axis_index("s")
    feat0 = ci * _FH
    start = si * _CPT

    def issue_load(j):
        row0 = (start + j) * _CH
        return pltpu.async_copy(
            nodes_hbm.at[pl.ds(row0, _CH), pl.ds(feat0, _FH)],
            rows_v.at[j % _D, pl.ds(0, _CH), :],
            load_sem,
        )

    loads = [issue_load(j) for j in range(_D)]
    loads += [None] * (_CPT - _D)
    scats = [None] * _CPT

    pltpu.sync_copy(ids_hbm.at[pl.ds(start, _CPT), :], ids_v)

    @pl.when(si == 0)
    def _():
        pltpu.sync_copy(zeros_hbm, acc_sh.at[pl.ds(0, _S), :])

    plsc.subcore_barrier()

    for j in range(_CPT):
        if j >= _E and j - _E + _D < _CPT:
            scats[j - _E].wait()
            loads[j - _E + _D] = issue_load(j - _E + _D)
        loads[j].wait()
        scats[j] = pltpu.async_copy(
            rows_v.at[j % _D], acc_sh.at[ids_v.at[j]], scat_sem, add=True
        )

    for j in range(_CPT - _D, _CPT):
        scats[j].wait()

    plsc.subcore_barrier()

    seg0 = si * (_S // _NS)
    pltpu.sync_copy(
        acc_sh.at[pl.ds(seg0, _S // _NS), :],
        out_hbm.at[pl.ds(seg0, _S // _NS), pl.ds(feat0, _FH)],
    )


def kernel(nodes, segment_ids):
    ids2d = segment_ids.reshape(_NS * _CPT, _CH)
    pad = jnp.full((_NS * _CPT, _CW - _CH), _S, jnp.int32)
    ids2d = jnp.concatenate([ids2d, pad], axis=1)
    zeros = jnp.zeros((_S, _FH), jnp.float32)
    mesh = plsc.VectorSubcoreMesh(core_axis_name="c", subcore_axis_name="s")
    f = pl.kernel(
        _body,
        out_type=jax.ShapeDtypeStruct((_S, _F), jnp.float32),
        mesh=mesh,
        scratch_types=[
            pltpu.VMEM((_CPT, _CW), jnp.int32),
            pltpu.VMEM((_D, _CW, _FH), jnp.float32),
            pltpu.VMEM_SHARED((_S + 1, _FH), jnp.float32),
            pltpu.SemaphoreType.DMA,
            pltpu.SemaphoreType.DMA,
        ],
        compiler_params=pltpu.CompilerParams(use_tc_tiling_on_sc=False),
    )
    return f(nodes, ids2d, zeros)


# speedup vs baseline: 6.3611x; 1.5304x over previous
import jax
import jax.numpy as jnp
from jax import lax
from jax.experimental import pallas as pl
from jax.experimental.pallas import tpu as pltpu
from jax.experimental.pallas import tpu_sc as plsc

_N = 100000
_F = 128
_S = 128
_CH = 125
_CW = 128
_NS = 16
_NC = 2
_CPT = 50
_PW = 64
_FH = _F // _NC
_D = 8
_E = 4
_L = 16
_UNROLL = 5


def _body(nodes_hbm, ids_hbm, pidx_hbm, zeros_hbm, out_hbm,
          ids_v, rows_v, psum_v, pidx_v, acc_sh, load_sem, scat_sem):
    ci = lax.axis_index("c")
    si = lax.axis_index("s")
    feat0 = ci * _FH
    start = si * _CPT

    def issue_load(j):
        row0 = (start + j) * _CH
        return pltpu.async_copy(
            nodes_hbm.at[pl.ds(row0, _CH), pl.ds(feat0, _FH)],
            rows_v.at[j % _D, pl.ds(0, _CH), :],
            load_sem,
        )

    loads = [issue_load(j) for j in range(_D)]
    loads += [None] * (_CPT - _D)
    scats = [None] * _CPT

    pltpu.sync_copy(ids_hbm.at[pl.ds(start, _CPT), :], ids_v)
    pltpu.sync_copy(pidx_hbm.at[si], pidx_v)

    @pl.when(si == 0)
    def _():
        pltpu.sync_copy(zeros_hbm, acc_sh.at[pl.ds(0, _S), :])

    plsc.subcore_barrier()

    def pflag(j):
        v = pidx_v[0, pl.ds((j // _L) * _L, _L)]
        return v[j % _L]

    def sum_pure_chunk(slot, j):
        def body(i, acc):
            a0, a1, a2, a3 = acc
            r0 = i * _UNROLL
            for u in range(_UNROLL):
                r = r0 + u
                a0 = a0 + rows_v[slot, r, pl.ds(0 * _L, _L)]
                a1 = a1 + rows_v[slot, r, pl.ds(1 * _L, _L)]
                a2 = a2 + rows_v[slot, r, pl.ds(2 * _L, _L)]
                a3 = a3 + rows_v[slot, r, pl.ds(3 * _L, _L)]
            return a0, a1, a2, a3

        z = jnp.zeros((_L,), jnp.float32)
        a0, a1, a2, a3 = lax.fori_loop(0, _CH // _UNROLL, body, (z, z, z, z))
        psum_v[j, pl.ds(0 * _L, _L)] = a0
        psum_v[j, pl.ds(1 * _L, _L)] = a1
        psum_v[j, pl.ds(2 * _L, _L)] = a2
        psum_v[j, pl.ds(3 * _L, _L)] = a3

    for j in range(_CPT):
        if j >= _E and j - _E + _D < _CPT:
            k = j - _E

            @pl.when(pflag(k) == _S)
            def _(k=k):
                scats[k].wait()

            loads[k + _D] = issue_load(k + _D)
        loads[j].wait()

        @pl.when(pflag(j) != _S)
        def _(j=j):
            sum_pure_chunk(j % _D, j)

        @pl.when(pflag(j) == _S)
        def _(j=j):
            scats[j] = pltpu.async_copy(
                rows_v.at[j % _D], acc_sh.at[ids_v.at[j]], scat_sem,
                add=True,
            )

    for j in range(_CPT - _D, _CPT):
        @pl.when(pflag(j) == _S)
        def _(j=j):
            scats[j].wait()

    pltpu.sync_copy(psum_v, acc_sh.at[pidx_v.at[0]], add=True)

    plsc.subcore_barrier()

    seg0 = si * (_S // _NS)
    pltpu.sync_copy(
        acc_sh.at[pl.ds(seg0, _S // _NS), :],
        out_hbm.at[pl.ds(seg0, _S // _NS), pl.ds(feat0, _FH)],
    )


def kernel(nodes, segment_ids):
    ids_rs = segment_ids.reshape(_NS * _CPT, _CH)
    pad = jnp.full((_NS * _CPT, _CW - _CH), _S, jnp.int32)
    ids2d = jnp.concatenate([ids_rs, pad], axis=1)
    first = ids_rs[:, :1]
    pure = jnp.all(ids_rs == first, axis=1)
    pidx = jnp.where(pure, first[:, 0], _S).astype(jnp.int32)
    pidx = pidx.reshape(_NS, _CPT)
    pidx = jnp.concatenate(
        [pidx, jnp.full((_NS, _PW - _CPT), _S, jnp.int32)], axis=1
    ).reshape(_NS, 1, _PW)
    zeros = jnp.zeros((_S, _FH), jnp.float32)
    mesh = plsc.VectorSubcoreMesh(core_axis_name="c", subcore_axis_name="s")
    f = pl.kernel(
        _body,
        out_type=jax.ShapeDtypeStruct((_S, _F), jnp.float32),
        mesh=mesh,
        scratch_types=[
            pltpu.VMEM((_CPT, _CW), jnp.int32),
            pltpu.VMEM((_D, _CW, _FH), jnp.float32),
            pltpu.VMEM((_PW, _FH), jnp.float32),
            pltpu.VMEM((1, _PW), jnp.int32),
            pltpu.VMEM_SHARED((_S + 1, _FH), jnp.float32),
            pltpu.SemaphoreType.DMA,
            pltpu.SemaphoreType.DMA,
        ],
        compiler_params=pltpu.CompilerParams(use_tc_tiling_on_sc=False),
    )
    return f(nodes, ids2d, pidx, zeros)
